# Initial kernel scaffold; baseline (speedup 1.0000x reference)
import jax
import jax.numpy as jnp
from jax.experimental import pallas as pl


def kernel(x_in, params):
    raise NotImplementedError("write your pallas kernel here")


# speedup vs baseline: 74.7156x; 74.7156x over previous
import functools

import jax
import jax.numpy as jnp
from jax import lax
from jax.experimental import pallas as pl

H = 256
W = 512
C1, C2, C3 = 16, 12, 8
C12 = C1 * C2
C123 = C12 * C3
PAD1, PAD2 = 2, 4


def _bf(v):
    return v.astype(jnp.bfloat16)


def _leaky(v):
    return jnp.where(v >= 0, v, 0.01 * v)


def _argmax0(t, n):
    m = jnp.max(t, axis=0)
    io = lax.broadcasted_iota(jnp.int32, t.shape, 0)
    return jnp.min(jnp.where(t == m[None, :], io, n), axis=0)


def _dotT(wblk, oh):
    return lax.dot_general(
        _bf(wblk), _bf(oh), (((0,), (0,)), ((), ())),
        preferred_element_type=jnp.float32)


def _dense(wt, x):
    return lax.dot_general(
        _bf(wt), _bf(x), (((1,), (0,)), ((), ())),
        preferred_element_type=jnp.float32)


def _cond_stage(x, oh, wrefs, brefs, cins):
    u = x
    nl = len(cins)
    for i, cin in enumerate(cins):
        wg = _dotT(wrefs[i][0], oh)
        bg = _dotT(brefs[i][0], oh)
        ub = _bf(u).astype(jnp.float32)
        xe = jnp.broadcast_to(ub[:, None, :], (cin, 16, W))
        s = jnp.sum(xe * wg.reshape(cin, 16, W), axis=0) + bg
        u = _leaky(s) if i < nl - 1 else s
    return u


def _regressor_kernel(x_ref,
                      w10, w11, w12, b10, b11, b12,
                      w20, w21, w22, b20, b21, b22,
                      w30, w31, w32, b30, b31, b32,
                      out_ref):
    x = x_ref[0]
    xa = x[0:32]
    xb = x[32:64]
    xc = x[64:96]

    l = _dense(w10[0], xa) + b10[0]
    l = _leaky(l)
    l = _dense(w11[0], l) + b11[0]
    l = _leaky(l)
    l = _dense(w12[0], l) + b12[0]
    a = _argmax0(l, C1)

    oh2 = (lax.broadcasted_iota(jnp.int32, (C1, W), 0) == a[None, :]
           ).astype(jnp.float32)
    l2 = _cond_stage(xb, oh2, (w20, w21, w22), (b20, b21, b22), (32, 16, 16))
    b = _argmax0(l2, 16)
    c_unc = a * C2 + (b - PAD1)
    c = jnp.clip(c_unc, 0, C12 - 1)

    oh3 = (lax.broadcasted_iota(jnp.int32, (C12, W), 0) == c[None, :]
           ).astype(jnp.float32)
    l3 = _cond_stage(xc, oh3, (w30, w31, w32), (b30, b31, b32), (32, 16, 16))
    d = _argmax0(l3, 16)
    out_ref[0, 0, :] = jnp.clip(c_unc * C3 + (d - PAD2), 0, C123 - 1)


def kernel(x_in, params):
    xt = jnp.transpose(x_in[0], (1, 0, 2))
    w1 = params['w1']
    b1 = [bb.reshape(H, 16, 1) for bb in params['b1']]
    w2 = [ww.reshape(H, C1, ww.shape[1] * 16) for ww in params['w2']]
    b2 = [bb.reshape(H, C1, 16) for bb in params['b2']]
    w3 = [ww.reshape(H, C12, ww.shape[1] * 16) for ww in params['w3']]
    b3 = [bb.reshape(H, C12, 16) for bb in params['b3']]

    def im_line(h):
        return (h, 0, 0)

    def spec(arr):
        return pl.BlockSpec((1,) + arr.shape[1:], im_line)

    ins = [xt, w1[0], w1[1], w1[2], b1[0], b1[1], b1[2],
           w2[0], w2[1], w2[2], b2[0], b2[1], b2[2],
           w3[0], w3[1], w3[2], b3[0], b3[1], b3[2]]
    out = pl.pallas_call(
        _regressor_kernel,
        grid=(H,),
        in_specs=[spec(a) for a in ins],
        out_specs=pl.BlockSpec((1, 1, W), lambda h: (h, 0, 0)),
        out_shape=jax.ShapeDtypeStruct((H, 1, W), jnp.int32),
    )(*ins)
    return out.reshape(1, 1, H, W)


# speedup vs baseline: 80.2488x; 1.0741x over previous
import functools

import jax
import jax.numpy as jnp
from jax import lax
from jax.experimental import pallas as pl

H = 256
W = 512
C1, C2, C3 = 16, 12, 8
C12 = C1 * C2
C123 = C12 * C3
PAD1, PAD2 = 2, 4


def _bf(v):
    return v.astype(jnp.bfloat16)


def _leaky(v):
    return jnp.where(v >= 0, v, 0.01 * v)


def _argmax0(t, n):
    m = jnp.max(t, axis=0)
    io = lax.broadcasted_iota(jnp.int32, t.shape, 0)
    return jnp.min(jnp.where(t == m[None, :], io, n), axis=0)


def _dotT(wblk, oh):
    return lax.dot_general(
        _bf(wblk), _bf(oh), (((0,), (0,)), ((), ())),
        preferred_element_type=jnp.float32)


def _dense(wt, x):
    return lax.dot_general(
        _bf(wt), _bf(x), (((1,), (0,)), ((), ())),
        preferred_element_type=jnp.float32)


def _cond_stage(x, oh, wrefs, brefs, cins):
    u = x
    nl = len(cins)
    for i, cin in enumerate(cins):
        wg = _dotT(wrefs[i], oh)
        bg = _dotT(brefs[i], oh)
        ub = _bf(u).astype(jnp.float32)
        xe = jnp.broadcast_to(ub[:, None, :], (cin, 16, W))
        s = jnp.sum(xe * wg.reshape(cin, 16, W), axis=0) + bg
        u = _leaky(s) if i < nl - 1 else s
    return u


LINES_PER_STEP = 8


def _regressor_kernel(x_ref,
                      w10, w11, w12, b10, b11, b12,
                      w20, w21, w22, b20, b21, b22,
                      w30, w31, w32, b30, b31, b32,
                      out_ref):
    for j in range(LINES_PER_STEP):
        x = x_ref[0, :, j, :]
        xa = x[0:32]
        xb = x[32:64]
        xc = x[64:96]

        l = _dense(w10[j], xa) + b10[j]
        l = _leaky(l)
        l = _dense(w11[j], l) + b11[j]
        l = _leaky(l)
        l = _dense(w12[j], l) + b12[j]
        a = _argmax0(l, C1)

        oh2 = (lax.broadcasted_iota(jnp.int32, (C1, W), 0) == a[None, :]
               ).astype(jnp.float32)
        l2 = _cond_stage(xb, oh2, (w20[j], w21[j], w22[j]),
                         (b20[j], b21[j], b22[j]), (32, 16, 16))
        b = _argmax0(l2, 16)
        c_unc = a * C2 + (b - PAD1)
        c = jnp.clip(c_unc, 0, C12 - 1)

        oh3 = (lax.broadcasted_iota(jnp.int32, (C12, W), 0) == c[None, :]
               ).astype(jnp.float32)
        l3 = _cond_stage(xc, oh3, (w30[j], w31[j], w32[j]),
                         (b30[j], b31[j], b32[j]), (32, 16, 16))
        d = _argmax0(l3, 16)
        out_ref[j, :] = jnp.clip(c_unc * C3 + (d - PAD2), 0, C123 - 1)


def kernel(x_in, params):
    L = LINES_PER_STEP
    w1 = params['w1']
    b1 = [bb.reshape(H, 16, 1) for bb in params['b1']]
    w2 = [ww.reshape(H, C1, ww.shape[1] * 16) for ww in params['w2']]
    b2 = [bb.reshape(H, C1, 16) for bb in params['b2']]
    w3 = [ww.reshape(H, C12, ww.shape[1] * 16) for ww in params['w3']]
    b3 = [bb.reshape(H, C12, 16) for bb in params['b3']]

    def spec(arr):
        return pl.BlockSpec((L,) + arr.shape[1:], lambda h: (h, 0, 0))

    wins = [w1[0], w1[1], w1[2], b1[0], b1[1], b1[2],
            w2[0], w2[1], w2[2], b2[0], b2[1], b2[2],
            w3[0], w3[1], w3[2], b3[0], b3[1], b3[2]]
    out = pl.pallas_call(
        _regressor_kernel,
        grid=(H // L,),
        in_specs=[pl.BlockSpec((1, 96, L, W), lambda h: (0, 0, h, 0))]
                 + [spec(a) for a in wins],
        out_specs=pl.BlockSpec((L, W), lambda h: (h, 0)),
        out_shape=jax.ShapeDtypeStruct((H, W), jnp.int32),
    )(x_in, *wins)
    return out.reshape(1, 1, H, W)
